# Initial kernel scaffold; baseline (speedup 1.0000x reference)
import functools

import jax
import jax.numpy as jnp
from jax import lax
from jax.experimental import pallas as pl
from jax.experimental.pallas import tpu as pltpu
from jax.experimental.pallas import tpu_sc as plsc

N = 10000
E = 320000
IN_C = 128
HID = 256
H2 = HID // 2
NS = 16
NPT = N // NS
K = 100
CH = E // (K * NS)
R = 400
G = N // R
N16 = N // 16

_mesh = plsc.VectorSubcoreMesh(core_axis_name="c", subcore_axis_name="s")


def _hist_body(colv, iota, out0, out1, hist_v, idx_v, hist_sp, sem):
    c = lax.axis_index("c")
    s = lax.axis_index("s")
    rows_per_tile = (E // 16) // (2 * NS)
    base = (c * NS + s) * rows_per_tile
    pltpu.sync_copy(colv.at[pl.ds(base, rows_per_tile)], idx_v)

    def zero(i, carry):
        hist_v[i] = jnp.zeros((16,), jnp.float32)
        return carry
    lax.fori_loop(0, N16, zero, 0)

    ones = jnp.full((16,), 1.0, jnp.float32)

    def step(i, carry):
        iv = idx_v[i]
        plsc.addupdate_scatter(
            hist_v,
            [lax.shift_right_logical(iv, 4), jnp.bitwise_and(iv, 15)],
            ones,
        )
        return carry
    lax.fori_loop(0, rows_per_tile, step, 0)

    @pl.when(s == 0)
    def _():
        pltpu.sync_copy(hist_v, hist_sp)
    plsc.subcore_barrier()

    @pl.when(s != 0)
    def _():
        for k in range(5):
            pltpu.sync_copy(hist_v.at[pl.ds(125 * k, 125)],
                            hist_sp.at[iota.at[k]], add=True)
    plsc.subcore_barrier()

    @pl.when((s == 0) & (c == 0))
    def _():
        pltpu.sync_copy(hist_sp, out0)

    @pl.when((s == 0) & (c == 1))
    def _():
        pltpu.sync_copy(hist_sp, out1)


_hist_call = pl.kernel(
    _hist_body,
    out_type=[jax.ShapeDtypeStruct((N16, 16), jnp.float32),
              jax.ShapeDtypeStruct((N16, 16), jnp.float32)],
    mesh=_mesh,
    scratch_types=[
        pltpu.VMEM((N16, 16), jnp.float32),
        pltpu.VMEM((N16, 16), jnp.int32),
        pltpu.VMEM_SHARED((N16, 16), jnp.float32),
        pltpu.SemaphoreType.DMA,
    ],
)


def _iota_idx_vmem():
    return jnp.arange(N16, dtype=jnp.int32).reshape(5, 125)


def _pass_body(y0, y1, rowm, colm, out0, out1, acc, row_v, col_v, gbuf, sem):
    c = lax.axis_index("c")
    s = lax.axis_index("s")
    base = s * CH
    pltpu.sync_copy(rowm.at[pl.ds(base, CH)], row_v)
    pltpu.sync_copy(colm.at[pl.ds(base, CH)], col_v)

    r0 = s * NPT

    @pl.when(c == 0)
    def _():
        pltpu.sync_copy(y0.at[pl.ds(r0, NPT)], acc.at[pl.ds(r0, NPT)])

    @pl.when(c == 1)
    def _():
        pltpu.sync_copy(y1.at[pl.ds(r0, NPT)], acc.at[pl.ds(r0, NPT)])

    plsc.subcore_barrier()

    def step(j, carry):
        @pl.when(c == 0)
        def _():
            pltpu.async_copy(y0.at[row_v.at[j]], gbuf, sem)

        @pl.when(c == 1)
        def _():
            pltpu.async_copy(y1.at[row_v.at[j]], gbuf, sem)

        pltpu.make_async_copy(y0.at[row_v.at[j]], gbuf, sem).wait()
        pltpu.sync_copy(gbuf, acc.at[col_v.at[j]], add=True)
        return carry

    lax.fori_loop(0, CH, step, 0)
    plsc.subcore_barrier()

    @pl.when(c == 0)
    def _():
        pltpu.sync_copy(acc.at[pl.ds(r0, NPT)], out0.at[pl.ds(r0, NPT)])

    @pl.when(c == 1)
    def _():
        pltpu.sync_copy(acc.at[pl.ds(r0, NPT)], out1.at[pl.ds(r0, NPT)])


_pass_call = pl.kernel(
    _pass_body,
    out_type=[jax.ShapeDtypeStruct((N, H2), jnp.float32),
              jax.ShapeDtypeStruct((N, H2), jnp.float32)],
    mesh=_mesh,
    scratch_types=[
        pltpu.VMEM_SHARED((N, H2), jnp.float32),
        pltpu.VMEM((CH, K), jnp.int32),
        pltpu.VMEM((CH, K), jnp.int32),
        pltpu.VMEM((K, H2), jnp.float32),
        pltpu.SemaphoreType.DMA,
    ],
)


def _rsqrt_body(h0, h1, d_ref):
    d_ref[...] = lax.rsqrt(h0[...] + h1[...] + 1.0)


def _d_from_hist(h0, h1):
    d = pl.pallas_call(
        _rsqrt_body,
        out_shape=jax.ShapeDtypeStruct((N16, 16), jnp.float32),
    )(h0, h1)
    return d.reshape(N, 1)


def _matmul1(x, w1t, b1, d):
    def kern(x_r, wa_r, b_r, d_r, xl_r, y0_r, y1_r):
        xl = jnp.dot(x_r[...], wa_r[...],
                     preferred_element_type=jnp.float32) + b_r[...]
        xl_r[...] = xl
        y = d_r[...] * xl
        y0_r[...] = y[:, :H2]
        y1_r[...] = y[:, H2:]

    return pl.pallas_call(
        kern,
        grid=(G,),
        in_specs=[
            pl.BlockSpec((R, IN_C), lambda i: (i, 0)),
            pl.BlockSpec((IN_C, HID), lambda i: (0, 0)),
            pl.BlockSpec((1, HID), lambda i: (0, 0)),
            pl.BlockSpec((R, 1), lambda i: (i, 0)),
        ],
        out_specs=[
            pl.BlockSpec((R, HID), lambda i: (i, 0)),
            pl.BlockSpec((R, H2), lambda i: (i, 0)),
            pl.BlockSpec((R, H2), lambda i: (i, 0)),
        ],
        out_shape=[
            jax.ShapeDtypeStruct((N, HID), jnp.float32),
            jax.ShapeDtypeStruct((N, H2), jnp.float32),
            jax.ShapeDtypeStruct((N, H2), jnp.float32),
        ],
    )(x, w1t, b1, d)


def _matmul2(h0, h1, w2a, w2b, b2, d):
    def kern(h0_r, h1_r, wa_r, wb_r, b_r, d_r, xl_r, y0_r, y1_r):
        xl = (jnp.dot(h0_r[...], wa_r[...], preferred_element_type=jnp.float32)
              + jnp.dot(h1_r[...], wb_r[...],
                        preferred_element_type=jnp.float32)
              + b_r[...])
        xl_r[...] = xl
        y = d_r[...] * xl
        y0_r[...] = y[:, :H2]
        y1_r[...] = y[:, H2:]

    return pl.pallas_call(
        kern,
        grid=(G,),
        in_specs=[
            pl.BlockSpec((R, H2), lambda i: (i, 0)),
            pl.BlockSpec((R, H2), lambda i: (i, 0)),
            pl.BlockSpec((H2, HID), lambda i: (0, 0)),
            pl.BlockSpec((H2, HID), lambda i: (0, 0)),
            pl.BlockSpec((1, HID), lambda i: (0, 0)),
            pl.BlockSpec((R, 1), lambda i: (i, 0)),
        ],
        out_specs=[
            pl.BlockSpec((R, HID), lambda i: (i, 0)),
            pl.BlockSpec((R, H2), lambda i: (i, 0)),
            pl.BlockSpec((R, H2), lambda i: (i, 0)),
        ],
        out_shape=[
            jax.ShapeDtypeStruct((N, HID), jnp.float32),
            jax.ShapeDtypeStruct((N, H2), jnp.float32),
            jax.ShapeDtypeStruct((N, H2), jnp.float32),
        ],
    )(h0, h1, w2a, w2b, b2, d)


def _combine_mid(a0, a1, xl, d, m):
    def kern(a0_r, a1_r, xl_r, d_r, m_r, y0_r, y1_r):
        dd = d_r[...]
        mm = m_r[...]
        xl = xl_r[...]
        y0_r[...] = dd * ((dd * a0_r[...]) * mm + xl[:, :H2] * (1.0 - mm))
        y1_r[...] = dd * ((dd * a1_r[...]) * mm + xl[:, H2:] * (1.0 - mm))

    return pl.pallas_call(
        kern,
        grid=(G,),
        in_specs=[
            pl.BlockSpec((R, H2), lambda i: (i, 0)),
            pl.BlockSpec((R, H2), lambda i: (i, 0)),
            pl.BlockSpec((R, HID), lambda i: (i, 0)),
            pl.BlockSpec((R, 1), lambda i: (i, 0)),
            pl.BlockSpec((R, 1), lambda i: (i, 0)),
        ],
        out_specs=[
            pl.BlockSpec((R, H2), lambda i: (i, 0)),
            pl.BlockSpec((R, H2), lambda i: (i, 0)),
        ],
        out_shape=[
            jax.ShapeDtypeStruct((N, H2), jnp.float32),
            jax.ShapeDtypeStruct((N, H2), jnp.float32),
        ],
    )(a0, a1, xl, d, m)


def _combine_end(a0, a1, xl, d, m, bias, *, relu, split_out):
    def kern(a0_r, a1_r, xl_r, d_r, m_r, b_r, *outs):
        dd = d_r[...]
        mm = m_r[...]
        xl = xl_r[...]
        b = b_r[...]
        t0 = (dd * a0_r[...]) * (1.0 - mm) + xl[:, :H2] * mm + b[:, :H2]
        t1 = (dd * a1_r[...]) * (1.0 - mm) + xl[:, H2:] * mm + b[:, H2:]
        if relu:
            t0 = jnp.maximum(t0, 0.0)
            t1 = jnp.maximum(t1, 0.0)
        if split_out:
            outs[0][...] = t0
            outs[1][...] = t1
        else:
            outs[0][...] = jnp.concatenate([t0, t1], axis=1)

    if split_out:
        out_specs = [pl.BlockSpec((R, H2), lambda i: (i, 0)),
                     pl.BlockSpec((R, H2), lambda i: (i, 0))]
        out_shape = [jax.ShapeDtypeStruct((N, H2), jnp.float32),
                     jax.ShapeDtypeStruct((N, H2), jnp.float32)]
    else:
        out_specs = [pl.BlockSpec((R, HID), lambda i: (i, 0))]
        out_shape = [jax.ShapeDtypeStruct((N, HID), jnp.float32)]

    return pl.pallas_call(
        kern,
        grid=(G,),
        in_specs=[
            pl.BlockSpec((R, H2), lambda i: (i, 0)),
            pl.BlockSpec((R, H2), lambda i: (i, 0)),
            pl.BlockSpec((R, HID), lambda i: (i, 0)),
            pl.BlockSpec((R, 1), lambda i: (i, 0)),
            pl.BlockSpec((R, 1), lambda i: (i, 0)),
            pl.BlockSpec((1, HID), lambda i: (0, 0)),
        ],
        out_specs=out_specs,
        out_shape=out_shape,
    )(a0, a1, xl, d, m, bias)


def kernel(x, edge_index, M, W1, b1, bias1, W2, b2, bias2):
    row = edge_index[0]
    col = edge_index[1]
    rowm = row.reshape(E // K, K)
    colm = col.reshape(E // K, K)
    colv = col.reshape(E // 16, 16)
    mf = M.astype(jnp.float32)

    h0p, h1p = _hist_call(colv, _iota_idx_vmem())
    d = _d_from_hist(h0p, h1p)

    w1t = W1.T
    w2a = W2[:, :H2].T
    w2b = W2[:, H2:].T
    b1r = b1.reshape(1, HID)
    b2r = b2.reshape(1, HID)
    bias1r = bias1.reshape(1, HID)
    bias2r = bias2.reshape(1, HID)

    xl1, y0, y1 = _matmul1(x, w1t, b1r, d)
    a0, a1 = _pass_call(y0, y1, rowm, colm)
    y0b, y1b = _combine_mid(a0, a1, xl1, d, mf)
    a0b, a1b = _pass_call(y0b, y1b, rowm, colm)
    hh0, hh1 = _combine_end(a0b, a1b, xl1, d, mf, bias1r,
                            relu=True, split_out=True)

    xl2, z0, z1 = _matmul2(hh0, hh1, w2a, w2b, b2r, d)
    c0, c1 = _pass_call(z0, z1, rowm, colm)
    z0b, z1b = _combine_mid(c0, c1, xl2, d, mf)
    c0b, c1b = _pass_call(z0b, z1b, rowm, colm)
    (out,) = _combine_end(c0b, c1b, xl2, d, mf, bias2r,
                          relu=False, split_out=False)
    return out


# speedup vs baseline: 10.7448x; 10.7448x over previous
import jax
import jax.numpy as jnp
from jax import lax
from jax.experimental import pallas as pl
from jax.experimental.pallas import tpu as pltpu
from jax.experimental.pallas import tpu_sc as plsc

N = 10000
E = 320000
IN_C = 128
HID = 256
H2 = HID // 2
NS = 16
NP = 10240
NPT = NP // NS
K = 100
CH = E // (K * NS)
R = 400
G = N // R
HK = 100
HCH = E // (32 * HK)

_mesh = plsc.VectorSubcoreMesh(core_axis_name="c", subcore_axis_name="s")


def _hist_body(colv3, out, acc_sp, idx_v, ones_v, zeros_v, sem):
    c = lax.axis_index("c")
    s = lax.axis_index("s")
    w = c * NS + s
    pltpu.sync_copy(colv3.at[w], idx_v)

    def fill(i, carry):
        ones_v[pl.ds(i * 16, 16)] = jnp.full((16,), 1.0, jnp.float32)
        zeros_v[pl.ds(i * 16, 16)] = jnp.zeros((16,), jnp.float32)
        return carry
    lax.fori_loop(0, NPT // 16, fill, 0)

    r0 = s * NPT
    pltpu.sync_copy(zeros_v, acc_sp.at[pl.ds(r0, NPT)])
    plsc.subcore_barrier()

    def step(i, carry):
        pltpu.sync_copy(ones_v.at[pl.ds(0, HK)],
                        acc_sp.at[idx_v.at[i]], add=True)
        return carry
    lax.fori_loop(0, HCH, step, 0)
    plsc.subcore_barrier()

    @pl.when(c == 0)
    def _():
        pltpu.sync_copy(acc_sp.at[pl.ds(r0, NPT)], out.at[0, 0, pl.ds(r0, NPT)])

    @pl.when(c == 1)
    def _():
        pltpu.sync_copy(acc_sp.at[pl.ds(r0, NPT)], out.at[1, 0, pl.ds(r0, NPT)])


_hist_call = pl.kernel(
    _hist_body,
    out_type=jax.ShapeDtypeStruct((2, 1, NP), jnp.float32),
    mesh=_mesh,
    scratch_types=[
        pltpu.VMEM_SHARED((NP,), jnp.float32),
        pltpu.VMEM((HCH, HK), jnp.int32),
        pltpu.VMEM((NPT,), jnp.float32),
        pltpu.VMEM((NPT,), jnp.float32),
        pltpu.SemaphoreType.DMA,
    ],
)


def _pass_body(y0, y1, rowm, colm, out0, out1, acc, row_v, col_v, gbuf, sem):
    c = lax.axis_index("c")
    s = lax.axis_index("s")

    r0 = s * NPT

    @pl.when(c == 0)
    def _():
        pltpu.sync_copy(y0.at[pl.ds(r0, NPT)], acc.at[pl.ds(r0, NPT)])

    @pl.when(c == 1)
    def _():
        pltpu.sync_copy(y1.at[pl.ds(r0, NPT)], acc.at[pl.ds(r0, NPT)])

    plsc.subcore_barrier()

    def group(g, carry):
        t = s * (CH // 8) + g
        pltpu.sync_copy(rowm.at[t], row_v)
        pltpu.sync_copy(colm.at[t], col_v)
        for k in range(8):
            @pl.when(c == 0)
            def _():
                pltpu.async_copy(y0.at[row_v.at[k]], gbuf, sem)

            @pl.when(c == 1)
            def _():
                pltpu.async_copy(y1.at[row_v.at[k]], gbuf, sem)

            pltpu.make_async_copy(y0.at[row_v.at[k]], gbuf, sem).wait()
            pltpu.sync_copy(gbuf, acc.at[col_v.at[k]], add=True)
        return carry

    lax.fori_loop(0, CH // 8, group, 0)
    plsc.subcore_barrier()

    @pl.when(c == 0)
    def _():
        pltpu.sync_copy(acc.at[pl.ds(r0, NPT)], out0.at[pl.ds(r0, NPT)])

    @pl.when(c == 1)
    def _():
        pltpu.sync_copy(acc.at[pl.ds(r0, NPT)], out1.at[pl.ds(r0, NPT)])


_pass_call = pl.kernel(
    _pass_body,
    out_type=[jax.ShapeDtypeStruct((NP, H2), jnp.float32),
              jax.ShapeDtypeStruct((NP, H2), jnp.float32)],
    mesh=_mesh,
    scratch_types=[
        pltpu.VMEM_SHARED((NP, H2), jnp.float32),
        pltpu.VMEM((8, K), jnp.int32),
        pltpu.VMEM((8, K), jnp.int32),
        pltpu.VMEM((K, H2), jnp.float32),
        pltpu.SemaphoreType.DMA,
    ],
)


def _rsqrt_body(h_ref, d_ref):
    h = h_ref[...]
    d_ref[...] = lax.rsqrt(h[0] + h[1] + 1.0)


def _d_from_hist(hist):
    DB = 1024
    d = pl.pallas_call(
        _rsqrt_body,
        grid=(NP // DB,),
        in_specs=[pl.BlockSpec((2, DB, 1), lambda i: (0, i, 0))],
        out_specs=pl.BlockSpec((DB, 1), lambda i: (i, 0)),
        out_shape=jax.ShapeDtypeStruct((NP, 1), jnp.float32),
    )(hist.reshape(2, NP, 1))
    return d[:N]


def _matmul1(x, w1t, b1, d):
    def kern(x_r, wa_r, b_r, d_r, xl_r, y0_r, y1_r):
        xl = jnp.dot(x_r[...], wa_r[...],
                     preferred_element_type=jnp.float32) + b_r[...]
        xl_r[...] = xl
        y = d_r[...] * xl
        y0_r[...] = y[:, :H2]
        y1_r[...] = y[:, H2:]

    return pl.pallas_call(
        kern,
        grid=(G,),
        in_specs=[
            pl.BlockSpec((R, IN_C), lambda i: (i, 0)),
            pl.BlockSpec((IN_C, HID), lambda i: (0, 0)),
            pl.BlockSpec((1, HID), lambda i: (0, 0)),
            pl.BlockSpec((R, 1), lambda i: (i, 0)),
        ],
        out_specs=[
            pl.BlockSpec((R, HID), lambda i: (i, 0)),
            pl.BlockSpec((R, H2), lambda i: (i, 0)),
            pl.BlockSpec((R, H2), lambda i: (i, 0)),
        ],
        out_shape=[
            jax.ShapeDtypeStruct((N, HID), jnp.float32),
            jax.ShapeDtypeStruct((NP, H2), jnp.float32),
            jax.ShapeDtypeStruct((NP, H2), jnp.float32),
        ],
    )(x, w1t, b1, d)


def _matmul2(h0, h1, w2a, w2b, b2, d):
    def kern(h0_r, h1_r, wa_r, wb_r, b_r, d_r, xl_r, y0_r, y1_r):
        xl = (jnp.dot(h0_r[...], wa_r[...], preferred_element_type=jnp.float32)
              + jnp.dot(h1_r[...], wb_r[...],
                        preferred_element_type=jnp.float32)
              + b_r[...])
        xl_r[...] = xl
        y = d_r[...] * xl
        y0_r[...] = y[:, :H2]
        y1_r[...] = y[:, H2:]

    return pl.pallas_call(
        kern,
        grid=(G,),
        in_specs=[
            pl.BlockSpec((R, H2), lambda i: (i, 0)),
            pl.BlockSpec((R, H2), lambda i: (i, 0)),
            pl.BlockSpec((H2, HID), lambda i: (0, 0)),
            pl.BlockSpec((H2, HID), lambda i: (0, 0)),
            pl.BlockSpec((1, HID), lambda i: (0, 0)),
            pl.BlockSpec((R, 1), lambda i: (i, 0)),
        ],
        out_specs=[
            pl.BlockSpec((R, HID), lambda i: (i, 0)),
            pl.BlockSpec((R, H2), lambda i: (i, 0)),
            pl.BlockSpec((R, H2), lambda i: (i, 0)),
        ],
        out_shape=[
            jax.ShapeDtypeStruct((N, HID), jnp.float32),
            jax.ShapeDtypeStruct((NP, H2), jnp.float32),
            jax.ShapeDtypeStruct((NP, H2), jnp.float32),
        ],
    )(h0, h1, w2a, w2b, b2, d)


def _combine_mid(a0, a1, xl, d, m):
    def kern(a0_r, a1_r, xl_r, d_r, m_r, y0_r, y1_r):
        dd = d_r[...]
        mm = m_r[...]
        xl = xl_r[...]
        y0_r[...] = dd * ((dd * a0_r[...]) * mm + xl[:, :H2] * (1.0 - mm))
        y1_r[...] = dd * ((dd * a1_r[...]) * mm + xl[:, H2:] * (1.0 - mm))

    return pl.pallas_call(
        kern,
        grid=(G,),
        in_specs=[
            pl.BlockSpec((R, H2), lambda i: (i, 0)),
            pl.BlockSpec((R, H2), lambda i: (i, 0)),
            pl.BlockSpec((R, HID), lambda i: (i, 0)),
            pl.BlockSpec((R, 1), lambda i: (i, 0)),
            pl.BlockSpec((R, 1), lambda i: (i, 0)),
        ],
        out_specs=[
            pl.BlockSpec((R, H2), lambda i: (i, 0)),
            pl.BlockSpec((R, H2), lambda i: (i, 0)),
        ],
        out_shape=[
            jax.ShapeDtypeStruct((NP, H2), jnp.float32),
            jax.ShapeDtypeStruct((NP, H2), jnp.float32),
        ],
    )(a0, a1, xl, d, m)


def _combine_end(a0, a1, xl, d, m, bias, *, relu, split_out):
    def kern(a0_r, a1_r, xl_r, d_r, m_r, b_r, *outs):
        dd = d_r[...]
        mm = m_r[...]
        xl = xl_r[...]
        b = b_r[...]
        t0 = (dd * a0_r[...]) * (1.0 - mm) + xl[:, :H2] * mm + b[:, :H2]
        t1 = (dd * a1_r[...]) * (1.0 - mm) + xl[:, H2:] * mm + b[:, H2:]
        if relu:
            t0 = jnp.maximum(t0, 0.0)
            t1 = jnp.maximum(t1, 0.0)
        if split_out:
            outs[0][...] = t0
            outs[1][...] = t1
        else:
            outs[0][...] = jnp.concatenate([t0, t1], axis=1)

    if split_out:
        out_specs = [pl.BlockSpec((R, H2), lambda i: (i, 0)),
                     pl.BlockSpec((R, H2), lambda i: (i, 0))]
        out_shape = [jax.ShapeDtypeStruct((N, H2), jnp.float32),
                     jax.ShapeDtypeStruct((N, H2), jnp.float32)]
    else:
        out_specs = [pl.BlockSpec((R, HID), lambda i: (i, 0))]
        out_shape = [jax.ShapeDtypeStruct((N, HID), jnp.float32)]

    return pl.pallas_call(
        kern,
        grid=(G,),
        in_specs=[
            pl.BlockSpec((R, H2), lambda i: (i, 0)),
            pl.BlockSpec((R, H2), lambda i: (i, 0)),
            pl.BlockSpec((R, HID), lambda i: (i, 0)),
            pl.BlockSpec((R, 1), lambda i: (i, 0)),
            pl.BlockSpec((R, 1), lambda i: (i, 0)),
            pl.BlockSpec((1, HID), lambda i: (0, 0)),
        ],
        out_specs=out_specs,
        out_shape=out_shape,
    )(a0, a1, xl, d, m, bias)


def kernel(x, edge_index, M, W1, b1, bias1, W2, b2, bias2):
    row = edge_index[0]
    col = edge_index[1]
    rowm = row.reshape(E // (8 * K), 8, K)
    colm = col.reshape(E // (8 * K), 8, K)
    colv3 = col.reshape(32, HCH, HK)
    mf = M.astype(jnp.float32)

    hist = _hist_call(colv3)
    d = _d_from_hist(hist)

    w1t = W1.T
    w2a = W2[:, :H2].T
    w2b = W2[:, H2:].T
    b1r = b1.reshape(1, HID)
    b2r = b2.reshape(1, HID)
    bias1r = bias1.reshape(1, HID)
    bias2r = bias2.reshape(1, HID)

    xl1, y0, y1 = _matmul1(x, w1t, b1r, d)
    a0, a1 = _pass_call(y0, y1, rowm, colm)
    y0b, y1b = _combine_mid(a0, a1, xl1, d, mf)
    a0b, a1b = _pass_call(y0b, y1b, rowm, colm)
    hh0, hh1 = _combine_end(a0b, a1b, xl1, d, mf, bias1r,
                            relu=True, split_out=True)

    xl2, z0, z1 = _matmul2(hh0, hh1, w2a, w2b, b2r, d)
    c0, c1 = _pass_call(z0, z1, rowm, colm)
    z0b, z1b = _combine_mid(c0, c1, xl2, d, mf)
    c0b, c1b = _pass_call(z0b, z1b, rowm, colm)
    (out,) = _combine_end(c0b, c1b, xl2, d, mf, bias2r,
                          relu=False, split_out=False)
    return out


# speedup vs baseline: 17.1563x; 1.5967x over previous
import jax
import jax.numpy as jnp
from jax import lax
from jax.experimental import pallas as pl
from jax.experimental.pallas import tpu as pltpu
from jax.experimental.pallas import tpu_sc as plsc

N = 10000
E = 320000
IN_C = 128
HID = 256
H2 = HID // 2
NS = 16
NP = 10240
NPT = NP // NS
K = 100
CH = E // (K * NS)
R = 400
G = N // R
HK = 100
HCH = E // (32 * HK)

_mesh = plsc.VectorSubcoreMesh(core_axis_name="c", subcore_axis_name="s")


def _hist_body(colv3, out, acc_sp, idx_v, ones_v, zeros_v, sem):
    c = lax.axis_index("c")
    s = lax.axis_index("s")
    w = c * NS + s
    pltpu.sync_copy(colv3.at[w], idx_v)

    def fill(i, carry):
        ones_v[pl.ds(i * 16, 16)] = jnp.full((16,), 1.0, jnp.float32)
        zeros_v[pl.ds(i * 16, 16)] = jnp.zeros((16,), jnp.float32)
        return carry
    lax.fori_loop(0, NPT // 16, fill, 0)

    r0 = s * NPT
    pltpu.sync_copy(zeros_v, acc_sp.at[pl.ds(r0, NPT)])
    plsc.subcore_barrier()

    def step(i, carry):
        pltpu.sync_copy(ones_v.at[pl.ds(0, HK)],
                        acc_sp.at[idx_v.at[i]], add=True)
        return carry
    lax.fori_loop(0, HCH, step, 0)
    plsc.subcore_barrier()

    @pl.when(c == 0)
    def _():
        pltpu.sync_copy(acc_sp.at[pl.ds(r0, NPT)], out.at[0, 0, pl.ds(r0, NPT)])

    @pl.when(c == 1)
    def _():
        pltpu.sync_copy(acc_sp.at[pl.ds(r0, NPT)], out.at[1, 0, pl.ds(r0, NPT)])


_hist_call = pl.kernel(
    _hist_body,
    out_type=jax.ShapeDtypeStruct((2, 1, NP), jnp.float32),
    mesh=_mesh,
    scratch_types=[
        pltpu.VMEM_SHARED((NP,), jnp.float32),
        pltpu.VMEM((HCH, HK), jnp.int32),
        pltpu.VMEM((NPT,), jnp.float32),
        pltpu.VMEM((NPT,), jnp.float32),
        pltpu.SemaphoreType.DMA,
    ],
)


NBG = 20
NG = CH // NBG


def _pass_body(y0, y1, rowm, colm, out0, out1, acc,
               row_v, col_v, g0, g1, sg0, sg1, ss0, ss1):
    c = lax.axis_index("c")
    s = lax.axis_index("s")

    r0 = s * NPT

    @pl.when(c == 0)
    def _():
        pltpu.sync_copy(y0.at[pl.ds(r0, NPT)], acc.at[pl.ds(r0, NPT)])

    @pl.when(c == 1)
    def _():
        pltpu.sync_copy(y1.at[pl.ds(r0, NPT)], acc.at[pl.ds(r0, NPT)])

    plsc.subcore_barrier()

    gbufs = (g0, g1)
    gsems = (sg0, sg1)
    ssems = (ss0, ss1)

    def start_gather(idx_view, buf, sem):
        @pl.when(c == 0)
        def _():
            pltpu.async_copy(y0.at[idx_view], buf, sem)

        @pl.when(c == 1)
        def _():
            pltpu.async_copy(y1.at[idx_view], buf, sem)

    def wait_gather(buf, sem):
        pltpu.make_async_copy(y0.at[row_v.at[0]], buf, sem).wait()

    def wait_scatter(buf, sem):
        pltpu.make_async_copy(buf, acc.at[col_v.at[0]], sem).wait()

    def group(g, carry):
        t = s * NG + g
        pltpu.sync_copy(rowm.at[t], row_v)
        pltpu.sync_copy(colm.at[t], col_v)
        for k in range(NBG):
            p = k % 2
            q = (k + 1) % 2
            if k == 0:
                @pl.when(g > 0)
                def _():
                    wait_scatter(gbufs[0], ssems[0])
                start_gather(row_v.at[0], gbufs[0], gsems[0])
            if k + 1 < NBG:
                if k + 1 == 1:
                    @pl.when(g > 0)
                    def _():
                        wait_scatter(gbufs[1], ssems[1])
                else:
                    wait_scatter(gbufs[q], ssems[q])
                start_gather(row_v.at[k + 1], gbufs[q], gsems[q])
            wait_gather(gbufs[p], gsems[p])
            pltpu.async_copy(gbufs[p], acc.at[col_v.at[k]], ssems[p], add=True)
        return carry

    lax.fori_loop(0, NG, group, 0)
    wait_scatter(gbufs[0], ssems[0])
    wait_scatter(gbufs[1], ssems[1])
    plsc.subcore_barrier()

    @pl.when(c == 0)
    def _():
        pltpu.sync_copy(acc.at[pl.ds(r0, NPT)], out0.at[pl.ds(r0, NPT)])

    @pl.when(c == 1)
    def _():
        pltpu.sync_copy(acc.at[pl.ds(r0, NPT)], out1.at[pl.ds(r0, NPT)])


_pass_call = pl.kernel(
    _pass_body,
    out_type=[jax.ShapeDtypeStruct((NP, H2), jnp.float32),
              jax.ShapeDtypeStruct((NP, H2), jnp.float32)],
    mesh=_mesh,
    scratch_types=[
        pltpu.VMEM_SHARED((NP, H2), jnp.float32),
        pltpu.VMEM((NBG, K), jnp.int32),
        pltpu.VMEM((NBG, K), jnp.int32),
        pltpu.VMEM((K, H2), jnp.float32),
        pltpu.VMEM((K, H2), jnp.float32),
        pltpu.SemaphoreType.DMA,
        pltpu.SemaphoreType.DMA,
        pltpu.SemaphoreType.DMA,
        pltpu.SemaphoreType.DMA,
    ],
)


def _rsqrt_body(h_ref, d_ref):
    h = h_ref[...]
    d_ref[...] = lax.rsqrt(h[0] + h[1] + 1.0)


def _d_from_hist(hist):
    DB = 1024
    d = pl.pallas_call(
        _rsqrt_body,
        grid=(NP // DB,),
        in_specs=[pl.BlockSpec((2, DB, 1), lambda i: (0, i, 0))],
        out_specs=pl.BlockSpec((DB, 1), lambda i: (i, 0)),
        out_shape=jax.ShapeDtypeStruct((NP, 1), jnp.float32),
    )(hist.reshape(2, NP, 1))
    return d[:N]


def _matmul1(x, w1t, b1, d):
    def kern(x_r, wa_r, b_r, d_r, xl_r, y0_r, y1_r):
        xl = jnp.dot(x_r[...], wa_r[...],
                     preferred_element_type=jnp.float32) + b_r[...]
        xl_r[...] = xl
        y = d_r[...] * xl
        y0_r[...] = y[:, :H2]
        y1_r[...] = y[:, H2:]

    return pl.pallas_call(
        kern,
        grid=(G,),
        in_specs=[
            pl.BlockSpec((R, IN_C), lambda i: (i, 0)),
            pl.BlockSpec((IN_C, HID), lambda i: (0, 0)),
            pl.BlockSpec((1, HID), lambda i: (0, 0)),
            pl.BlockSpec((R, 1), lambda i: (i, 0)),
        ],
        out_specs=[
            pl.BlockSpec((R, HID), lambda i: (i, 0)),
            pl.BlockSpec((R, H2), lambda i: (i, 0)),
            pl.BlockSpec((R, H2), lambda i: (i, 0)),
        ],
        out_shape=[
            jax.ShapeDtypeStruct((N, HID), jnp.float32),
            jax.ShapeDtypeStruct((NP, H2), jnp.float32),
            jax.ShapeDtypeStruct((NP, H2), jnp.float32),
        ],
    )(x, w1t, b1, d)


def _matmul2(h0, h1, w2a, w2b, b2, d):
    def kern(h0_r, h1_r, wa_r, wb_r, b_r, d_r, xl_r, y0_r, y1_r):
        xl = (jnp.dot(h0_r[...], wa_r[...], preferred_element_type=jnp.float32)
              + jnp.dot(h1_r[...], wb_r[...],
                        preferred_element_type=jnp.float32)
              + b_r[...])
        xl_r[...] = xl
        y = d_r[...] * xl
        y0_r[...] = y[:, :H2]
        y1_r[...] = y[:, H2:]

    return pl.pallas_call(
        kern,
        grid=(G,),
        in_specs=[
            pl.BlockSpec((R, H2), lambda i: (i, 0)),
            pl.BlockSpec((R, H2), lambda i: (i, 0)),
            pl.BlockSpec((H2, HID), lambda i: (0, 0)),
            pl.BlockSpec((H2, HID), lambda i: (0, 0)),
            pl.BlockSpec((1, HID), lambda i: (0, 0)),
            pl.BlockSpec((R, 1), lambda i: (i, 0)),
        ],
        out_specs=[
            pl.BlockSpec((R, HID), lambda i: (i, 0)),
            pl.BlockSpec((R, H2), lambda i: (i, 0)),
            pl.BlockSpec((R, H2), lambda i: (i, 0)),
        ],
        out_shape=[
            jax.ShapeDtypeStruct((N, HID), jnp.float32),
            jax.ShapeDtypeStruct((NP, H2), jnp.float32),
            jax.ShapeDtypeStruct((NP, H2), jnp.float32),
        ],
    )(h0, h1, w2a, w2b, b2, d)


def _combine_mid(a0, a1, xl, d, m):
    def kern(a0_r, a1_r, xl_r, d_r, m_r, y0_r, y1_r):
        dd = d_r[...]
        mm = m_r[...]
        xl = xl_r[...]
        y0_r[...] = dd * ((dd * a0_r[...]) * mm + xl[:, :H2] * (1.0 - mm))
        y1_r[...] = dd * ((dd * a1_r[...]) * mm + xl[:, H2:] * (1.0 - mm))

    return pl.pallas_call(
        kern,
        grid=(G,),
        in_specs=[
            pl.BlockSpec((R, H2), lambda i: (i, 0)),
            pl.BlockSpec((R, H2), lambda i: (i, 0)),
            pl.BlockSpec((R, HID), lambda i: (i, 0)),
            pl.BlockSpec((R, 1), lambda i: (i, 0)),
            pl.BlockSpec((R, 1), lambda i: (i, 0)),
        ],
        out_specs=[
            pl.BlockSpec((R, H2), lambda i: (i, 0)),
            pl.BlockSpec((R, H2), lambda i: (i, 0)),
        ],
        out_shape=[
            jax.ShapeDtypeStruct((NP, H2), jnp.float32),
            jax.ShapeDtypeStruct((NP, H2), jnp.float32),
        ],
    )(a0, a1, xl, d, m)


def _combine_end(a0, a1, xl, d, m, bias, *, relu, split_out):
    def kern(a0_r, a1_r, xl_r, d_r, m_r, b_r, *outs):
        dd = d_r[...]
        mm = m_r[...]
        xl = xl_r[...]
        b = b_r[...]
        t0 = (dd * a0_r[...]) * (1.0 - mm) + xl[:, :H2] * mm + b[:, :H2]
        t1 = (dd * a1_r[...]) * (1.0 - mm) + xl[:, H2:] * mm + b[:, H2:]
        if relu:
            t0 = jnp.maximum(t0, 0.0)
            t1 = jnp.maximum(t1, 0.0)
        if split_out:
            outs[0][...] = t0
            outs[1][...] = t1
        else:
            outs[0][...] = jnp.concatenate([t0, t1], axis=1)

    if split_out:
        out_specs = [pl.BlockSpec((R, H2), lambda i: (i, 0)),
                     pl.BlockSpec((R, H2), lambda i: (i, 0))]
        out_shape = [jax.ShapeDtypeStruct((N, H2), jnp.float32),
                     jax.ShapeDtypeStruct((N, H2), jnp.float32)]
    else:
        out_specs = [pl.BlockSpec((R, HID), lambda i: (i, 0))]
        out_shape = [jax.ShapeDtypeStruct((N, HID), jnp.float32)]

    return pl.pallas_call(
        kern,
        grid=(G,),
        in_specs=[
            pl.BlockSpec((R, H2), lambda i: (i, 0)),
            pl.BlockSpec((R, H2), lambda i: (i, 0)),
            pl.BlockSpec((R, HID), lambda i: (i, 0)),
            pl.BlockSpec((R, 1), lambda i: (i, 0)),
            pl.BlockSpec((R, 1), lambda i: (i, 0)),
            pl.BlockSpec((1, HID), lambda i: (0, 0)),
        ],
        out_specs=out_specs,
        out_shape=out_shape,
    )(a0, a1, xl, d, m, bias)


def kernel(x, edge_index, M, W1, b1, bias1, W2, b2, bias2):
    row = edge_index[0]
    col = edge_index[1]
    rowm = row.reshape(E // (NBG * K), NBG, K)
    colm = col.reshape(E // (NBG * K), NBG, K)
    colv3 = col.reshape(32, HCH, HK)
    mf = M.astype(jnp.float32)

    hist = _hist_call(colv3)
    d = _d_from_hist(hist)

    w1t = W1.T
    w2a = W2[:, :H2].T
    w2b = W2[:, H2:].T
    b1r = b1.reshape(1, HID)
    b2r = b2.reshape(1, HID)
    bias1r = bias1.reshape(1, HID)
    bias2r = bias2.reshape(1, HID)

    xl1, y0, y1 = _matmul1(x, w1t, b1r, d)
    a0, a1 = _pass_call(y0, y1, rowm, colm)
    y0b, y1b = _combine_mid(a0, a1, xl1, d, mf)
    a0b, a1b = _pass_call(y0b, y1b, rowm, colm)
    hh0, hh1 = _combine_end(a0b, a1b, xl1, d, mf, bias1r,
                            relu=True, split_out=True)

    xl2, z0, z1 = _matmul2(hh0, hh1, w2a, w2b, b2r, d)
    c0, c1 = _pass_call(z0, z1, rowm, colm)
    z0b, z1b = _combine_mid(c0, c1, xl2, d, mf)
    c0b, c1b = _pass_call(z0b, z1b, rowm, colm)
    (out,) = _combine_end(c0b, c1b, xl2, d, mf, bias2r,
                          relu=False, split_out=False)
    return out


# speedup vs baseline: 18.2195x; 1.0620x over previous
import jax
import jax.numpy as jnp
from jax import lax
from jax.experimental import pallas as pl
from jax.experimental.pallas import tpu as pltpu
from jax.experimental.pallas import tpu_sc as plsc

N = 10000
E = 320000
IN_C = 128
HID = 256
H2 = HID // 2
NS = 16
NP = 10240
NPT = NP // NS
K = 125
CH = E // (K * NS)
NBG = 16
NG = CH // NBG
R = 400
G = N // R
_mesh = plsc.VectorSubcoreMesh(core_axis_name="c", subcore_axis_name="s")


HK = 100
HCH = E // (32 * HK)


def _hist_body(colv3, out, acc_sp, idx_v, ones_v, zeros_v, sem):
    c = lax.axis_index("c")
    s = lax.axis_index("s")
    w = c * NS + s
    pltpu.sync_copy(colv3.at[w], idx_v)

    def fill(i, carry):
        ones_v[pl.ds(i * 16, 16)] = jnp.full((16,), 1.0, jnp.float32)
        zeros_v[pl.ds(i * 16, 16)] = jnp.zeros((16,), jnp.float32)
        return carry
    lax.fori_loop(0, NPT // 16, fill, 0)

    r0 = s * NPT
    pltpu.sync_copy(zeros_v, acc_sp.at[pl.ds(r0, NPT)])
    plsc.subcore_barrier()

    def step(i, carry):
        pltpu.sync_copy(ones_v.at[pl.ds(0, HK)],
                        acc_sp.at[idx_v.at[i]], add=True)
        return carry
    lax.fori_loop(0, HCH, step, 0)
    plsc.subcore_barrier()

    @pl.when(c == 0)
    def _():
        pltpu.sync_copy(acc_sp.at[pl.ds(r0, NPT)], out.at[0, 0, pl.ds(r0, NPT)])

    @pl.when(c == 1)
    def _():
        pltpu.sync_copy(acc_sp.at[pl.ds(r0, NPT)], out.at[1, 0, pl.ds(r0, NPT)])


_hist_call = pl.kernel(
    _hist_body,
    out_type=jax.ShapeDtypeStruct((2, 1, NP), jnp.float32),
    mesh=_mesh,
    scratch_types=[
        pltpu.VMEM_SHARED((NP,), jnp.float32),
        pltpu.VMEM((HCH, HK), jnp.int32),
        pltpu.VMEM((NPT,), jnp.float32),
        pltpu.VMEM((NPT,), jnp.float32),
        pltpu.SemaphoreType.DMA,
    ],
)


def _pass_body(y0, y1, rowm, colm, out0, out1, acc,
               row_v, col_v, g0, g1, sg0, sg1, ss0, ss1):
    c = lax.axis_index("c")
    s = lax.axis_index("s")

    r0 = s * NPT

    @pl.when(c == 0)
    def _():
        pltpu.sync_copy(y0.at[pl.ds(r0, NPT)], acc.at[pl.ds(r0, NPT)])

    @pl.when(c == 1)
    def _():
        pltpu.sync_copy(y1.at[pl.ds(r0, NPT)], acc.at[pl.ds(r0, NPT)])

    plsc.subcore_barrier()

    gbufs = (g0, g1)
    gsems = (sg0, sg1)
    ssems = (ss0, ss1)

    def start_gather(idx_view, buf, sem):
        @pl.when(c == 0)
        def _():
            pltpu.async_copy(y0.at[idx_view], buf, sem)

        @pl.when(c == 1)
        def _():
            pltpu.async_copy(y1.at[idx_view], buf, sem)

    def wait_gather(buf, sem):
        pltpu.make_async_copy(y0.at[row_v.at[0]], buf, sem).wait()

    def wait_scatter(buf, sem):
        pltpu.make_async_copy(buf, acc.at[col_v.at[0]], sem).wait()

    def group(g, carry):
        off = pl.multiple_of((s * NG + g) * NBG, 8)
        pltpu.sync_copy(rowm.at[pl.ds(off, NBG)], row_v)
        pltpu.sync_copy(colm.at[pl.ds(off, NBG)], col_v)
        for k in range(NBG):
            p = k % 2
            q = (k + 1) % 2
            if k == 0:
                @pl.when(g > 0)
                def _():
                    wait_scatter(gbufs[0], ssems[0])
                start_gather(row_v.at[0], gbufs[0], gsems[0])
            if k + 1 < NBG:
                if k + 1 == 1:
                    @pl.when(g > 0)
                    def _():
                        wait_scatter(gbufs[1], ssems[1])
                else:
                    wait_scatter(gbufs[q], ssems[q])
                start_gather(row_v.at[k + 1], gbufs[q], gsems[q])
            wait_gather(gbufs[p], gsems[p])
            pltpu.async_copy(gbufs[p], acc.at[col_v.at[k]], ssems[p], add=True)
        return carry

    lax.fori_loop(0, NG, group, 0)
    wait_scatter(gbufs[0], ssems[0])
    wait_scatter(gbufs[1], ssems[1])
    plsc.subcore_barrier()

    @pl.when(c == 0)
    def _():
        pltpu.sync_copy(acc.at[pl.ds(r0, NPT)], out0.at[pl.ds(r0, NPT)])

    @pl.when(c == 1)
    def _():
        pltpu.sync_copy(acc.at[pl.ds(r0, NPT)], out1.at[pl.ds(r0, NPT)])


_pass_call = pl.kernel(
    _pass_body,
    out_type=[jax.ShapeDtypeStruct((NP, H2), jnp.float32),
              jax.ShapeDtypeStruct((NP, H2), jnp.float32)],
    mesh=_mesh,
    scratch_types=[
        pltpu.VMEM_SHARED((NP, H2), jnp.float32),
        pltpu.VMEM((NBG, K), jnp.int32),
        pltpu.VMEM((NBG, K), jnp.int32),
        pltpu.VMEM((K, H2), jnp.float32),
        pltpu.VMEM((K, H2), jnp.float32),
        pltpu.SemaphoreType.DMA,
        pltpu.SemaphoreType.DMA,
        pltpu.SemaphoreType.DMA,
        pltpu.SemaphoreType.DMA,
    ],
)


def _deg_spec():
    return pl.BlockSpec((2, R, 1), lambda i: (0, i, 0))


def _dvec(h_blk):
    return lax.rsqrt(h_blk[0] + h_blk[1] + 1.0)


_DNUMS = (((1,), (1,)), ((), ()))


def _matmul1(x, w1, b1, deg):
    def kern(x_r, w_r, b_r, h_r, xl_r, y0_r, y1_r):
        xl = lax.dot_general(x_r[...], w_r[...], _DNUMS,
                             preferred_element_type=jnp.float32) + b_r[...]
        xl_r[...] = xl
        y = _dvec(h_r[...]) * xl
        y0_r[...] = y[:, :H2]
        y1_r[...] = y[:, H2:]

    return pl.pallas_call(
        kern,
        grid=(G,),
        in_specs=[
            pl.BlockSpec((R, IN_C), lambda i: (i, 0)),
            pl.BlockSpec((HID, IN_C), lambda i: (0, 0)),
            pl.BlockSpec((1, HID), lambda i: (0, 0)),
            _deg_spec(),
        ],
        out_specs=[
            pl.BlockSpec((R, HID), lambda i: (i, 0)),
            pl.BlockSpec((R, H2), lambda i: (i, 0)),
            pl.BlockSpec((R, H2), lambda i: (i, 0)),
        ],
        out_shape=[
            jax.ShapeDtypeStruct((N, HID), jnp.float32),
            jax.ShapeDtypeStruct((NP, H2), jnp.float32),
            jax.ShapeDtypeStruct((NP, H2), jnp.float32),
        ],
    )(x, w1, b1, deg)


def _combine_mid(a0, a1, xl, deg, m):
    def kern(a0_r, a1_r, xl_r, h_r, m_r, y0_r, y1_r):
        dd = _dvec(h_r[...])
        mm = m_r[...]
        xl = xl_r[...]
        y0_r[...] = dd * ((dd * a0_r[...]) * mm + xl[:, :H2] * (1.0 - mm))
        y1_r[...] = dd * ((dd * a1_r[...]) * mm + xl[:, H2:] * (1.0 - mm))

    return pl.pallas_call(
        kern,
        grid=(G,),
        in_specs=[
            pl.BlockSpec((R, H2), lambda i: (i, 0)),
            pl.BlockSpec((R, H2), lambda i: (i, 0)),
            pl.BlockSpec((R, HID), lambda i: (i, 0)),
            _deg_spec(),
            pl.BlockSpec((R, 1), lambda i: (i, 0)),
        ],
        out_specs=[
            pl.BlockSpec((R, H2), lambda i: (i, 0)),
            pl.BlockSpec((R, H2), lambda i: (i, 0)),
        ],
        out_shape=[
            jax.ShapeDtypeStruct((NP, H2), jnp.float32),
            jax.ShapeDtypeStruct((NP, H2), jnp.float32),
        ],
    )(a0, a1, xl, deg, m)


def _layer2_head(a0, a1, xl1, deg, m, bias1, w2, b2):
    def kern(a0_r, a1_r, xl_r, h_r, m_r, b_r, w_r, b2_r, xl2_r, z0_r, z1_r):
        dd = _dvec(h_r[...])
        mm = m_r[...]
        xl = xl_r[...]
        b = b_r[...]
        t0 = (dd * a0_r[...]) * (1.0 - mm) + xl[:, :H2] * mm + b[:, :H2]
        t1 = (dd * a1_r[...]) * (1.0 - mm) + xl[:, H2:] * mm + b[:, H2:]
        t0 = jnp.maximum(t0, 0.0)
        t1 = jnp.maximum(t1, 0.0)
        w = w_r[...]
        xl2 = (lax.dot_general(t0, w[:, :H2], _DNUMS,
                               preferred_element_type=jnp.float32)
               + lax.dot_general(t1, w[:, H2:], _DNUMS,
                                 preferred_element_type=jnp.float32)
               + b2_r[...])
        xl2_r[...] = xl2
        y = dd * xl2
        z0_r[...] = y[:, :H2]
        z1_r[...] = y[:, H2:]

    return pl.pallas_call(
        kern,
        grid=(G,),
        in_specs=[
            pl.BlockSpec((R, H2), lambda i: (i, 0)),
            pl.BlockSpec((R, H2), lambda i: (i, 0)),
            pl.BlockSpec((R, HID), lambda i: (i, 0)),
            _deg_spec(),
            pl.BlockSpec((R, 1), lambda i: (i, 0)),
            pl.BlockSpec((1, HID), lambda i: (0, 0)),
            pl.BlockSpec((HID, HID), lambda i: (0, 0)),
            pl.BlockSpec((1, HID), lambda i: (0, 0)),
        ],
        out_specs=[
            pl.BlockSpec((R, HID), lambda i: (i, 0)),
            pl.BlockSpec((R, H2), lambda i: (i, 0)),
            pl.BlockSpec((R, H2), lambda i: (i, 0)),
        ],
        out_shape=[
            jax.ShapeDtypeStruct((N, HID), jnp.float32),
            jax.ShapeDtypeStruct((NP, H2), jnp.float32),
            jax.ShapeDtypeStruct((NP, H2), jnp.float32),
        ],
    )(a0, a1, xl1, deg, m, bias1, w2, b2)


def _combine_final(a0, a1, xl, deg, m, bias):
    def kern(a0_r, a1_r, xl_r, h_r, m_r, b_r, o_r):
        dd = _dvec(h_r[...])
        mm = m_r[...]
        xl = xl_r[...]
        b = b_r[...]
        t0 = (dd * a0_r[...]) * (1.0 - mm) + xl[:, :H2] * mm + b[:, :H2]
        t1 = (dd * a1_r[...]) * (1.0 - mm) + xl[:, H2:] * mm + b[:, H2:]
        o_r[...] = jnp.concatenate([t0, t1], axis=1)

    return pl.pallas_call(
        kern,
        grid=(G,),
        in_specs=[
            pl.BlockSpec((R, H2), lambda i: (i, 0)),
            pl.BlockSpec((R, H2), lambda i: (i, 0)),
            pl.BlockSpec((R, HID), lambda i: (i, 0)),
            _deg_spec(),
            pl.BlockSpec((R, 1), lambda i: (i, 0)),
            pl.BlockSpec((1, HID), lambda i: (0, 0)),
        ],
        out_specs=pl.BlockSpec((R, HID), lambda i: (i, 0)),
        out_shape=jax.ShapeDtypeStruct((N, HID), jnp.float32),
    )(a0, a1, xl, deg, m, bias)


def kernel(x, edge_index, M, W1, b1, bias1, W2, b2, bias2):
    row = edge_index[0]
    col = edge_index[1]
    rowm = row.reshape(E // K, K)
    colm = col.reshape(E // K, K)
    mf = M.astype(jnp.float32)

    colv3 = col.reshape(32, HCH, HK)
    deg = _hist_call(colv3).reshape(2, NP, 1)

    b1r = b1.reshape(1, HID)
    b2r = b2.reshape(1, HID)
    bias1r = bias1.reshape(1, HID)
    bias2r = bias2.reshape(1, HID)

    xl1, y0, y1 = _matmul1(x, W1, b1r, deg)
    a0, a1 = _pass_call(y0, y1, rowm, colm)
    y0b, y1b = _combine_mid(a0, a1, xl1, deg, mf)
    a0b, a1b = _pass_call(y0b, y1b, rowm, colm)

    xl2, z0, z1 = _layer2_head(a0b, a1b, xl1, deg, mf, bias1r, W2, b2r)

    c0, c1 = _pass_call(z0, z1, rowm, colm)
    z0b, z1b = _combine_mid(c0, c1, xl2, deg, mf)
    c0b, c1b = _pass_call(z0b, z1b, rowm, colm)
    return _combine_final(c0b, c1b, xl2, deg, mf, bias2r)


# speedup vs baseline: 19.1632x; 1.0518x over previous
import jax
import jax.numpy as jnp
from jax import lax
from jax.experimental import pallas as pl
from jax.experimental.pallas import tpu as pltpu
from jax.experimental.pallas import tpu_sc as plsc

N = 10000
E = 320000
IN_C = 128
HID = 256
H2 = HID // 2
NS = 16
NP = 10240
NPT = NP // NS
K = 125
CH = E // (K * NS)
NBG = 16
NG = CH // NBG
R = 2000
G = N // R
_mesh = plsc.VectorSubcoreMesh(core_axis_name="c", subcore_axis_name="s")


HK = 100
HCH = E // (32 * HK)


def _hist_body(colv3, out, acc_sp, idx_v, ones_v, zeros_v, sem):
    c = lax.axis_index("c")
    s = lax.axis_index("s")
    w = c * NS + s
    pltpu.sync_copy(colv3.at[w], idx_v)

    def fill(i, carry):
        ones_v[pl.ds(i * 16, 16)] = jnp.full((16,), 1.0, jnp.float32)
        zeros_v[pl.ds(i * 16, 16)] = jnp.zeros((16,), jnp.float32)
        return carry
    lax.fori_loop(0, NPT // 16, fill, 0)

    r0 = s * NPT
    pltpu.sync_copy(zeros_v, acc_sp.at[pl.ds(r0, NPT)])
    plsc.subcore_barrier()

    def step(i, carry):
        pltpu.sync_copy(ones_v.at[pl.ds(0, HK)],
                        acc_sp.at[idx_v.at[i]], add=True)
        return carry
    lax.fori_loop(0, HCH, step, 0)
    plsc.subcore_barrier()

    @pl.when(c == 0)
    def _():
        pltpu.sync_copy(acc_sp.at[pl.ds(r0, NPT)], out.at[0, 0, pl.ds(r0, NPT)])

    @pl.when(c == 1)
    def _():
        pltpu.sync_copy(acc_sp.at[pl.ds(r0, NPT)], out.at[1, 0, pl.ds(r0, NPT)])


_hist_call = pl.kernel(
    _hist_body,
    out_type=jax.ShapeDtypeStruct((2, 1, NP), jnp.float32),
    mesh=_mesh,
    scratch_types=[
        pltpu.VMEM_SHARED((NP,), jnp.float32),
        pltpu.VMEM((HCH, HK), jnp.int32),
        pltpu.VMEM((NPT,), jnp.float32),
        pltpu.VMEM((NPT,), jnp.float32),
        pltpu.SemaphoreType.DMA,
    ],
)


def _pass_body(y0, y1, rowm, colm, out0, out1, acc,
               row_v, col_v, g0, g1, sg0, sg1, ss0, ss1):
    c = lax.axis_index("c")
    s = lax.axis_index("s")

    r0 = s * NPT

    @pl.when(c == 0)
    def _():
        pltpu.sync_copy(y0.at[pl.ds(r0, NPT)], acc.at[pl.ds(r0, NPT)])

    @pl.when(c == 1)
    def _():
        pltpu.sync_copy(y1.at[pl.ds(r0, NPT)], acc.at[pl.ds(r0, NPT)])

    plsc.subcore_barrier()

    gbufs = (g0, g1)
    gsems = (sg0, sg1)
    ssems = (ss0, ss1)

    def start_gather(idx_view, buf, sem):
        @pl.when(c == 0)
        def _():
            pltpu.async_copy(y0.at[idx_view], buf, sem)

        @pl.when(c == 1)
        def _():
            pltpu.async_copy(y1.at[idx_view], buf, sem)

    def wait_gather(buf, sem):
        pltpu.make_async_copy(y0.at[row_v.at[0]], buf, sem).wait()

    def wait_scatter(buf, sem):
        pltpu.make_async_copy(buf, acc.at[col_v.at[0]], sem).wait()

    def group(g, carry):
        off = pl.multiple_of((s * NG + g) * NBG, 8)
        pltpu.sync_copy(rowm.at[pl.ds(off, NBG)], row_v)
        pltpu.sync_copy(colm.at[pl.ds(off, NBG)], col_v)
        for k in range(NBG):
            p = k % 2
            q = (k + 1) % 2
            if k == 0:
                @pl.when(g > 0)
                def _():
                    wait_scatter(gbufs[0], ssems[0])
                start_gather(row_v.at[0], gbufs[0], gsems[0])
            if k + 1 < NBG:
                if k + 1 == 1:
                    @pl.when(g > 0)
                    def _():
                        wait_scatter(gbufs[1], ssems[1])
                else:
                    wait_scatter(gbufs[q], ssems[q])
                start_gather(row_v.at[k + 1], gbufs[q], gsems[q])
            wait_gather(gbufs[p], gsems[p])
            pltpu.async_copy(gbufs[p], acc.at[col_v.at[k]], ssems[p], add=True)
        return carry

    lax.fori_loop(0, NG, group, 0)
    wait_scatter(gbufs[0], ssems[0])
    wait_scatter(gbufs[1], ssems[1])
    plsc.subcore_barrier()

    @pl.when(c == 0)
    def _():
        pltpu.sync_copy(acc.at[pl.ds(r0, NPT)], out0.at[pl.ds(r0, NPT)])

    @pl.when(c == 1)
    def _():
        pltpu.sync_copy(acc.at[pl.ds(r0, NPT)], out1.at[pl.ds(r0, NPT)])


_pass_call = pl.kernel(
    _pass_body,
    out_type=[jax.ShapeDtypeStruct((NP, H2), jnp.float32),
              jax.ShapeDtypeStruct((NP, H2), jnp.float32)],
    mesh=_mesh,
    scratch_types=[
        pltpu.VMEM_SHARED((NP, H2), jnp.float32),
        pltpu.VMEM((NBG, K), jnp.int32),
        pltpu.VMEM((NBG, K), jnp.int32),
        pltpu.VMEM((K, H2), jnp.float32),
        pltpu.VMEM((K, H2), jnp.float32),
        pltpu.SemaphoreType.DMA,
        pltpu.SemaphoreType.DMA,
        pltpu.SemaphoreType.DMA,
        pltpu.SemaphoreType.DMA,
    ],
)


def _deg_spec():
    return pl.BlockSpec((2, R, 1), lambda i: (0, i, 0))


def _dvec(h_blk):
    return lax.rsqrt(h_blk[0] + h_blk[1] + 1.0)


_DNUMS = (((1,), (1,)), ((), ()))


def _matmul1(x, w1, b1, deg):
    def kern(x_r, w_r, b_r, h_r, xl_r, y0_r, y1_r):
        xl = lax.dot_general(x_r[...], w_r[...], _DNUMS,
                             preferred_element_type=jnp.float32) + b_r[...]
        xl_r[...] = xl
        y = _dvec(h_r[...]) * xl
        y0_r[...] = y[:, :H2]
        y1_r[...] = y[:, H2:]

    return pl.pallas_call(
        kern,
        grid=(G,),
        in_specs=[
            pl.BlockSpec((R, IN_C), lambda i: (i, 0)),
            pl.BlockSpec((HID, IN_C), lambda i: (0, 0)),
            pl.BlockSpec((1, HID), lambda i: (0, 0)),
            _deg_spec(),
        ],
        out_specs=[
            pl.BlockSpec((R, HID), lambda i: (i, 0)),
            pl.BlockSpec((R, H2), lambda i: (i, 0)),
            pl.BlockSpec((R, H2), lambda i: (i, 0)),
        ],
        out_shape=[
            jax.ShapeDtypeStruct((N, HID), jnp.float32),
            jax.ShapeDtypeStruct((NP, H2), jnp.float32),
            jax.ShapeDtypeStruct((NP, H2), jnp.float32),
        ],
    )(x, w1, b1, deg)


def _combine_mid(a0, a1, xl, deg, m):
    def kern(a0_r, a1_r, xl_r, h_r, m_r, y0_r, y1_r):
        dd = _dvec(h_r[...])
        mm = m_r[...]
        xl = xl_r[...]
        y0_r[...] = dd * jnp.where(mm, dd * a0_r[...], xl[:, :H2])
        y1_r[...] = dd * jnp.where(mm, dd * a1_r[...], xl[:, H2:])

    return pl.pallas_call(
        kern,
        grid=(G,),
        in_specs=[
            pl.BlockSpec((R, H2), lambda i: (i, 0)),
            pl.BlockSpec((R, H2), lambda i: (i, 0)),
            pl.BlockSpec((R, HID), lambda i: (i, 0)),
            _deg_spec(),
            pl.BlockSpec((R, 1), lambda i: (i, 0)),
        ],
        out_specs=[
            pl.BlockSpec((R, H2), lambda i: (i, 0)),
            pl.BlockSpec((R, H2), lambda i: (i, 0)),
        ],
        out_shape=[
            jax.ShapeDtypeStruct((NP, H2), jnp.float32),
            jax.ShapeDtypeStruct((NP, H2), jnp.float32),
        ],
    )(a0, a1, xl, deg, m)


def _layer2_head(a0, a1, xl1, deg, m, bias1, w2, b2):
    def kern(a0_r, a1_r, xl_r, h_r, m_r, b_r, w_r, b2_r, xl2_r, z0_r, z1_r):
        dd = _dvec(h_r[...])
        mm = m_r[...]
        xl = xl_r[...]
        b = b_r[...]
        t0 = jnp.where(mm, xl[:, :H2], dd * a0_r[...]) + b[:, :H2]
        t1 = jnp.where(mm, xl[:, H2:], dd * a1_r[...]) + b[:, H2:]
        t0 = jnp.maximum(t0, 0.0)
        t1 = jnp.maximum(t1, 0.0)
        w = w_r[...]
        xl2 = (lax.dot_general(t0, w[:, :H2], _DNUMS,
                               preferred_element_type=jnp.float32)
               + lax.dot_general(t1, w[:, H2:], _DNUMS,
                                 preferred_element_type=jnp.float32)
               + b2_r[...])
        xl2_r[...] = xl2
        y = dd * xl2
        z0_r[...] = y[:, :H2]
        z1_r[...] = y[:, H2:]

    return pl.pallas_call(
        kern,
        grid=(G,),
        in_specs=[
            pl.BlockSpec((R, H2), lambda i: (i, 0)),
            pl.BlockSpec((R, H2), lambda i: (i, 0)),
            pl.BlockSpec((R, HID), lambda i: (i, 0)),
            _deg_spec(),
            pl.BlockSpec((R, 1), lambda i: (i, 0)),
            pl.BlockSpec((1, HID), lambda i: (0, 0)),
            pl.BlockSpec((HID, HID), lambda i: (0, 0)),
            pl.BlockSpec((1, HID), lambda i: (0, 0)),
        ],
        out_specs=[
            pl.BlockSpec((R, HID), lambda i: (i, 0)),
            pl.BlockSpec((R, H2), lambda i: (i, 0)),
            pl.BlockSpec((R, H2), lambda i: (i, 0)),
        ],
        out_shape=[
            jax.ShapeDtypeStruct((N, HID), jnp.float32),
            jax.ShapeDtypeStruct((NP, H2), jnp.float32),
            jax.ShapeDtypeStruct((NP, H2), jnp.float32),
        ],
    )(a0, a1, xl1, deg, m, bias1, w2, b2)


def _combine_final(a0, a1, xl, deg, m, bias):
    def kern(a0_r, a1_r, xl_r, h_r, m_r, b_r, o_r):
        dd = _dvec(h_r[...])
        mm = m_r[...]
        xl = xl_r[...]
        b = b_r[...]
        t0 = jnp.where(mm, xl[:, :H2], dd * a0_r[...]) + b[:, :H2]
        t1 = jnp.where(mm, xl[:, H2:], dd * a1_r[...]) + b[:, H2:]
        o_r[...] = jnp.concatenate([t0, t1], axis=1)

    return pl.pallas_call(
        kern,
        grid=(G,),
        in_specs=[
            pl.BlockSpec((R, H2), lambda i: (i, 0)),
            pl.BlockSpec((R, H2), lambda i: (i, 0)),
            pl.BlockSpec((R, HID), lambda i: (i, 0)),
            _deg_spec(),
            pl.BlockSpec((R, 1), lambda i: (i, 0)),
            pl.BlockSpec((1, HID), lambda i: (0, 0)),
        ],
        out_specs=pl.BlockSpec((R, HID), lambda i: (i, 0)),
        out_shape=jax.ShapeDtypeStruct((N, HID), jnp.float32),
    )(a0, a1, xl, deg, m, bias)


def kernel(x, edge_index, M, W1, b1, bias1, W2, b2, bias2):
    row = edge_index[0]
    col = edge_index[1]
    rowm = row.reshape(E // K, K)
    colm = col.reshape(E // K, K)
    colv3 = col.reshape(32, HCH, HK)
    deg = _hist_call(colv3).reshape(2, NP, 1)

    b1r = b1.reshape(1, HID)
    b2r = b2.reshape(1, HID)
    bias1r = bias1.reshape(1, HID)
    bias2r = bias2.reshape(1, HID)

    xl1, y0, y1 = _matmul1(x, W1, b1r, deg)
    a0, a1 = _pass_call(y0, y1, rowm, colm)
    y0b, y1b = _combine_mid(a0, a1, xl1, deg, M)
    a0b, a1b = _pass_call(y0b, y1b, rowm, colm)

    xl2, z0, z1 = _layer2_head(a0b, a1b, xl1, deg, M, bias1r, W2, b2r)

    c0, c1 = _pass_call(z0, z1, rowm, colm)
    z0b, z1b = _combine_mid(c0, c1, xl2, deg, M)
    c0b, c1b = _pass_call(z0b, z1b, rowm, colm)
    return _combine_final(c0b, c1b, xl2, deg, M, bias2r)
